# Initial kernel scaffold; baseline (speedup 1.0000x reference)
import jax
import jax.numpy as jnp
from jax.experimental import pallas as pl


def kernel(features, batch_ids):
    raise NotImplementedError("write your pallas kernel here")


# speedup vs baseline: 6.8569x; 6.8569x over previous
import jax
import jax.numpy as jnp
from jax.experimental import pallas as pl
from jax.experimental.pallas import tpu as pltpu

_NUM_SEG = 16
_BLOCK_ROWS = 2048


def _seg_max_body(ids_ref, x_ref, o_ref):
    i = pl.program_id(0)

    @pl.when(i == 0)
    def _init():
        o_ref[...] = jnp.full_like(o_ref, -jnp.inf)

    ids = ids_ref[...]
    x = x_ref[...]
    id_lo = ids[0, 0]
    id_hi = ids[_BLOCK_ROWS - 1, 0]
    for s in range(_NUM_SEG):
        @pl.when((id_lo <= s) & (s <= id_hi))
        def _do(s=s):
            m = ids == s
            col = jnp.max(jnp.where(m, x, -jnp.inf), axis=0)
            o_ref[s, :] = jnp.maximum(o_ref[s, :], col)


def kernel(features, batch_ids):
    n, d = features.shape
    nblk = n // _BLOCK_ROWS
    ids2 = batch_ids.astype(jnp.int32).reshape(n, 1)
    return pl.pallas_call(
        _seg_max_body,
        grid=(nblk,),
        in_specs=[
            pl.BlockSpec((_BLOCK_ROWS, 1), lambda i: (i, 0)),
            pl.BlockSpec((_BLOCK_ROWS, d), lambda i: (i, 0)),
        ],
        out_specs=pl.BlockSpec((_NUM_SEG, d), lambda i: (0, 0)),
        out_shape=jax.ShapeDtypeStruct((_NUM_SEG, d), jnp.float32),
    )(ids2, features)
